# baseline (speedup 1.0000x reference)
import functools

import jax
import jax.numpy as jnp
from jax import lax
from jax.experimental import pallas as pl
from jax.experimental.pallas import tpu as pltpu
from jax.experimental.pallas import tpu_sc as plsc

N, M = 10000, 32
ORIG, NBR, F = 128, 16, 64
F2 = 2 * F

BLK = 400
MH = M // 2
ROWSH = BLK * MH
GCH = 1000

_PREC = lax.Precision.DEFAULT


def _dot(a, b):
    return lax.dot_general(a, b, (((1,), (0,)), ((), ())),
                           precision=_PREC, preferred_element_type=jnp.float32)


def _sc_gather(table, idx_flat):
    n_idx = idx_flat.shape[0]
    mesh = plsc.VectorSubcoreMesh(core_axis_name="c", subcore_axis_name="s")
    n_workers = 32
    per_w = n_idx // n_workers

    @functools.partial(
        pl.kernel,
        mesh=mesh,
        out_type=jax.ShapeDtypeStruct((n_idx, F2), jnp.float32),
        scratch_types=[
            pltpu.VMEM((GCH,), jnp.int32),
            pltpu.VMEM((GCH, F2), jnp.float32),
            pltpu.SemaphoreType.DMA,
        ],
    )
    def gk(table_hbm, idx_hbm, out_hbm, idx_v, rows_v, sem):
        wid = lax.axis_index("s") * 2 + lax.axis_index("c")
        base = wid * per_w

        @pl.loop(0, per_w, step=GCH)
        def _(off):
            pltpu.sync_copy(idx_hbm.at[pl.ds(base + off, GCH)], idx_v)
            pltpu.async_copy(table_hbm.at[idx_v], rows_v, sem).wait()
            pltpu.sync_copy(rows_v, out_hbm.at[pl.ds(base + off, GCH)])

    return gk(table, idx_flat)


def _embed_body(a_ref, w_ref, b_ref, wn_ref, x_ref, y_ref):
    x = _dot(a_ref[...], w_ref[...]) + b_ref[...]
    x_ref[...] = x
    y_ref[...] = _dot(x, wn_ref[...])


def _embed(atom_fea, W_emb, b_emb, Wn0):
    return pl.pallas_call(
        _embed_body,
        grid=(N // 1000,),
        in_specs=[
            pl.BlockSpec((1000, ORIG), lambda i: (i, 0)),
            pl.BlockSpec((ORIG, F), lambda i: (0, 0)),
            pl.BlockSpec((1, F), lambda i: (0, 0)),
            pl.BlockSpec((F, F2), lambda i: (0, 0)),
        ],
        out_specs=[
            pl.BlockSpec((1000, F), lambda i: (i, 0)),
            pl.BlockSpec((1000, F2), lambda i: (i, 0)),
        ],
        out_shape=[
            jax.ShapeDtypeStruct((N, F), jnp.float32),
            jax.ShapeDtypeStruct((N, F2), jnp.float32),
        ],
    )(atom_fea, W_emb, b_emb.reshape(1, F), Wn0)


def _pass_a_body(g_ref, e_ref, x_ref, ws_ref, we_ref, bf_ref, st_ref, gc_ref):
    i = pl.program_id(0)
    s = _dot(x_ref[...], ws_ref[...]) + bf_ref[...]
    eterm = _dot(e_ref[...].reshape(ROWSH, NBR), we_ref[...])
    gated = (g_ref[...].reshape(ROWSH, F2) + eterm).reshape(MH, BLK, F2)
    gated = gated + s[None, :, :]
    gc_ref[...] = gated.astype(jnp.bfloat16)
    flat = gated.reshape(ROWSH, F2)
    both = jnp.concatenate([flat, flat * flat], axis=1)
    upd = _dot(jnp.ones((8, ROWSH), jnp.float32), both)

    @pl.when(i == 0)
    def _():
        st_ref[...] = upd

    @pl.when(i > 0)
    def _():
        st_ref[...] = st_ref[...] + upd


def _pass_a(g_half, e, x, Ws, We, bf, mo):
    return pl.pallas_call(
        _pass_a_body,
        grid=(N // BLK,),
        in_specs=[
            pl.BlockSpec((MH, BLK, F2), lambda i: (0, i, 0)),
            pl.BlockSpec((MH, BLK, NBR), lambda i: (mo, i, 0)),
            pl.BlockSpec((BLK, F), lambda i: (i, 0)),
            pl.BlockSpec((F, F2), lambda i: (0, 0)),
            pl.BlockSpec((NBR, F2), lambda i: (0, 0)),
            pl.BlockSpec((1, F2), lambda i: (0, 0)),
        ],
        out_specs=[
            pl.BlockSpec((8, 2 * F2), lambda i: (0, 0)),
            pl.BlockSpec((MH, BLK, F2), lambda i: (0, i, 0)),
        ],
        out_shape=[
            jax.ShapeDtypeStruct((8, 2 * F2), jnp.float32),
            jax.ShapeDtypeStruct((MH, N, F2), jnp.bfloat16),
        ],
    )(g_half, e, x, Ws, We, bf.reshape(1, F2))


def _pass_b_body(gc1_ref, gc2_ref, scale_ref, shift_ref, sum_ref, st_ref):
    i = pl.program_id(0)

    def half(gc_ref):
        gated = (gc_ref[...].astype(jnp.float32) * scale_ref[...][None]
                 + shift_ref[...][None])
        u = gated[..., :F]
        v = gated[..., F:]
        sig = 1.0 / (1.0 + jnp.exp(-u))
        sp = jnp.maximum(v, 0.0) + jnp.log(1.0 + jnp.exp(-jnp.abs(v)))
        return jnp.sum(sig * sp, axis=0)

    summed = half(gc1_ref) + half(gc2_ref)
    sum_ref[...] = summed
    both = jnp.concatenate([summed, summed * summed], axis=1)
    upd = _dot(jnp.ones((8, BLK), jnp.float32), both)

    @pl.when(i == 0)
    def _():
        st_ref[...] = upd

    @pl.when(i > 0)
    def _():
        st_ref[...] = st_ref[...] + upd


def _pass_b(gc1, gc2, scale, shift):
    return pl.pallas_call(
        _pass_b_body,
        grid=(N // BLK,),
        in_specs=[
            pl.BlockSpec((MH, BLK, F2), lambda i: (0, i, 0)),
            pl.BlockSpec((MH, BLK, F2), lambda i: (0, i, 0)),
            pl.BlockSpec((1, F2), lambda i: (0, 0)),
            pl.BlockSpec((1, F2), lambda i: (0, 0)),
        ],
        out_specs=[
            pl.BlockSpec((BLK, F), lambda i: (i, 0)),
            pl.BlockSpec((8, F2), lambda i: (0, 0)),
        ],
        out_shape=[
            jax.ShapeDtypeStruct((N, F), jnp.float32),
            jax.ShapeDtypeStruct((8, F2), jnp.float32),
        ],
    )(gc1, gc2, scale.reshape(1, F2), shift.reshape(1, F2))


def _update_body(x_ref, sm_ref, a_ref, c_ref, wn_ref, o_ref, y_ref):
    xn = jax.nn.softplus(x_ref[...] + sm_ref[...] * a_ref[...] + c_ref[...])
    o_ref[...] = xn
    y_ref[...] = _dot(xn, wn_ref[...])


def _update(x, summed, a, c, Wn_next):
    return pl.pallas_call(
        _update_body,
        grid=(N // 1000,),
        in_specs=[
            pl.BlockSpec((1000, F), lambda i: (i, 0)),
            pl.BlockSpec((1000, F), lambda i: (i, 0)),
            pl.BlockSpec((1, F), lambda i: (0, 0)),
            pl.BlockSpec((1, F), lambda i: (0, 0)),
            pl.BlockSpec((F, F2), lambda i: (0, 0)),
        ],
        out_specs=[
            pl.BlockSpec((1000, F), lambda i: (i, 0)),
            pl.BlockSpec((1000, F2), lambda i: (i, 0)),
        ],
        out_shape=[
            jax.ShapeDtypeStruct((N, F), jnp.float32),
            jax.ShapeDtypeStruct((N, F2), jnp.float32),
        ],
    )(x, summed, a.reshape(1, F), c.reshape(1, F), Wn_next)


def _update_head_body(x_ref, sm_ref, a_ref, c_ref, wh_ref, bh_ref, o_ref):
    xn = jax.nn.softplus(x_ref[...] + sm_ref[...] * a_ref[...] + c_ref[...])
    o_ref[...] = _dot(xn, wh_ref[...]) + bh_ref[...]


def _update_head(x, summed, a, c, W_head, b_head):
    H = W_head.shape[1]
    return pl.pallas_call(
        _update_head_body,
        grid=(N // 1000,),
        in_specs=[
            pl.BlockSpec((1000, F), lambda i: (i, 0)),
            pl.BlockSpec((1000, F), lambda i: (i, 0)),
            pl.BlockSpec((1, F), lambda i: (0, 0)),
            pl.BlockSpec((1, F), lambda i: (0, 0)),
            pl.BlockSpec((F, H), lambda i: (0, 0)),
            pl.BlockSpec((1, H), lambda i: (0, 0)),
        ],
        out_specs=pl.BlockSpec((1000, H), lambda i: (i, 0)),
        out_shape=jax.ShapeDtypeStruct((N, H), jnp.float32),
    )(x, summed, a.reshape(1, F), c.reshape(1, F), W_head,
      b_head.reshape(1, H))


def _conv_layer(x, y, idx1, idx2, e, Wf, bf, g1, b1, g2n, b2n):
    Ws, We = Wf[:F], Wf[F2:]
    We_bf = We.astype(jnp.bfloat16)
    ga = _sc_gather(y, idx1).reshape(MH, N, F2)
    sta, gca = _pass_a(ga, e, x, Ws, We_bf, bf, 0)
    gb = _sc_gather(y, idx2).reshape(MH, N, F2)
    stb, gcb = _pass_a(gb, e, x, Ws, We_bf, bf, 1)
    st = sta + stb

    mean = st[0, :F2] / (N * M)
    var = st[0, F2:] / (N * M) - mean * mean
    scale = g1 * lax.rsqrt(var + 1e-5)
    shift = b1 - mean * scale

    summed, st2 = _pass_b(gca, gcb, scale, shift)
    m2 = st2[0, :F] / N
    v2 = st2[0, F:] / N - m2 * m2
    a = g2n * lax.rsqrt(v2 + 1e-5)
    c = b2n - m2 * a
    return summed, a, c


def kernel(atom_fea, nbr_fea, nbr_fea_idx, crystal_atom_idx, W_emb, b_emb,
           conv0_Wf, conv0_bf, conv0_g1, conv0_b1, conv0_g2, conv0_b2,
           conv1_Wf, conv1_bf, conv1_g1, conv1_b1, conv1_g2, conv1_b2,
           conv2_Wf, conv2_bf, conv2_g1, conv2_b1, conv2_g2, conv2_b2,
           W_head, b_head):
    del crystal_atom_idx
    idx_t = nbr_fea_idx.T
    idx1 = idx_t[:MH].reshape(N * MH)
    idx2 = idx_t[MH:].reshape(N * MH)
    e = nbr_fea.transpose(1, 0, 2).astype(jnp.bfloat16)

    x, y = _embed(atom_fea, W_emb, b_emb, conv0_Wf[F:F2])
    params = [
        (conv0_Wf, conv0_bf, conv0_g1, conv0_b1, conv0_g2, conv0_b2),
        (conv1_Wf, conv1_bf, conv1_g1, conv1_b1, conv1_g2, conv1_b2),
        (conv2_Wf, conv2_bf, conv2_g1, conv2_b1, conv2_g2, conv2_b2),
    ]
    for l, p in enumerate(params):
        summed, a, c = _conv_layer(x, y, idx1, idx2, e, *p)
        if l < 2:
            x, y = _update(x, summed, a, c, params[l + 1][0][F:F2])
        else:
            return _update_head(x, summed, a, c, W_head, b_head)


# speedup vs baseline: 1.0497x; 1.0497x over previous
import functools

import jax
import jax.numpy as jnp
from jax import lax
from jax.experimental import pallas as pl
from jax.experimental.pallas import tpu as pltpu
from jax.experimental.pallas import tpu_sc as plsc

N, M = 10000, 32
ORIG, NBR, F = 128, 16, 64
F2 = 2 * F

BLK = 400
MH = M // 2
ROWSH = BLK * MH
GCH = 1000

_PREC = lax.Precision.DEFAULT


def _dot(a, b):
    return lax.dot_general(a, b, (((1,), (0,)), ((), ())),
                           precision=_PREC, preferred_element_type=jnp.float32)


def _sc_gather(table, idx_flat):
    n_idx = idx_flat.shape[0]
    mesh = plsc.VectorSubcoreMesh(core_axis_name="c", subcore_axis_name="s")
    n_workers = 32
    per_w = n_idx // n_workers

    @functools.partial(
        pl.kernel,
        mesh=mesh,
        out_type=jax.ShapeDtypeStruct((n_idx, F2), jnp.float32),
        scratch_types=[
            pltpu.VMEM((GCH,), jnp.int32),
            pltpu.VMEM((GCH, F2), jnp.float32),
            pltpu.SemaphoreType.DMA,
        ],
    )
    def gk(table_hbm, idx_hbm, out_hbm, idx_v, rows_v, sem):
        wid = lax.axis_index("s") * 2 + lax.axis_index("c")
        base = wid * per_w

        @pl.loop(0, per_w, step=GCH)
        def _(off):
            pltpu.sync_copy(idx_hbm.at[pl.ds(base + off, GCH)], idx_v)
            pltpu.async_copy(table_hbm.at[idx_v], rows_v, sem).wait()
            pltpu.sync_copy(rows_v, out_hbm.at[pl.ds(base + off, GCH)])

    return gk(table, idx_flat)


def _embed_body(a_ref, w_ref, b_ref, wn_ref, x_ref, y_ref):
    x = _dot(a_ref[...], w_ref[...]) + b_ref[...]
    x_ref[...] = x
    y_ref[...] = _dot(x, wn_ref[...])


def _embed(atom_fea, W_emb, b_emb, Wn0):
    return pl.pallas_call(
        _embed_body,
        grid=(N // 1000,),
        in_specs=[
            pl.BlockSpec((1000, ORIG), lambda i: (i, 0)),
            pl.BlockSpec((ORIG, F), lambda i: (0, 0)),
            pl.BlockSpec((1, F), lambda i: (0, 0)),
            pl.BlockSpec((F, F2), lambda i: (0, 0)),
        ],
        out_specs=[
            pl.BlockSpec((1000, F), lambda i: (i, 0)),
            pl.BlockSpec((1000, F2), lambda i: (i, 0)),
        ],
        out_shape=[
            jax.ShapeDtypeStruct((N, F), jnp.float32),
            jax.ShapeDtypeStruct((N, F2), jnp.float32),
        ],
    )(atom_fea, W_emb, b_emb.reshape(1, F), Wn0)


def _pass_a_body(g_ref, e_ref, x_ref, ws_ref, we_ref, bf_ref, st_ref):
    i = pl.program_id(0)
    s = _dot(x_ref[...], ws_ref[...]) + bf_ref[...]
    eterm = _dot(e_ref[...].reshape(ROWSH, NBR), we_ref[...])
    gated = (g_ref[...].reshape(ROWSH, F2) + eterm).reshape(MH, BLK, F2)
    gated = gated + s[None, :, :]
    flat = gated.reshape(ROWSH, F2)
    both = jnp.concatenate([flat, flat * flat], axis=1)
    upd = _dot(jnp.ones((8, ROWSH), jnp.float32), both)

    @pl.when(i == 0)
    def _():
        st_ref[...] = upd

    @pl.when(i > 0)
    def _():
        st_ref[...] = st_ref[...] + upd


def _pass_a(g_half, e, x, Ws, We, bf, mo):
    return pl.pallas_call(
        _pass_a_body,
        grid=(N // BLK,),
        in_specs=[
            pl.BlockSpec((MH, BLK, F2), lambda i: (0, i, 0)),
            pl.BlockSpec((MH, BLK, NBR), lambda i: (mo, i, 0)),
            pl.BlockSpec((BLK, F), lambda i: (i, 0)),
            pl.BlockSpec((F, F2), lambda i: (0, 0)),
            pl.BlockSpec((NBR, F2), lambda i: (0, 0)),
            pl.BlockSpec((1, F2), lambda i: (0, 0)),
        ],
        out_specs=pl.BlockSpec((8, 2 * F2), lambda i: (0, 0)),
        out_shape=jax.ShapeDtypeStruct((8, 2 * F2), jnp.float32),
    )(g_half, e, x, Ws, We, bf.reshape(1, F2))


def _pass_b_body(g1_ref, g2_ref, e_ref, x_ref, ws_ref, we_ref, bias_ref,
                 scale_ref, sum_ref, st_ref):
    i = pl.program_id(0)
    s = _dot(x_ref[...], ws_ref[...]) + bias_ref[...]
    ef = e_ref[...]

    def half(g_half, e_half):
        eterm = _dot(e_half.reshape(ROWSH, NBR), we_ref[...])
        gated = (g_half.reshape(ROWSH, F2) * scale_ref[...]
                 + eterm).reshape(MH, BLK, F2)
        gated = gated + s[None, :, :]
        u = gated[..., :F]
        v = gated[..., F:]
        sig = 0.5 * jnp.tanh(0.5 * u) + 0.5
        sp = jnp.maximum(v, 0.0) + jnp.log(1.0 + jnp.exp(-jnp.abs(v)))
        return jnp.sum(sig * sp, axis=0)

    summed = half(g1_ref[...], ef[:MH]) + half(g2_ref[...], ef[MH:])
    sum_ref[...] = summed
    both = jnp.concatenate([summed, summed * summed], axis=1)
    upd = _dot(jnp.ones((8, BLK), jnp.float32), both)

    @pl.when(i == 0)
    def _():
        st_ref[...] = upd

    @pl.when(i > 0)
    def _():
        st_ref[...] = st_ref[...] + upd


def _pass_b(g1, g2, e, x, Ws_f, We_f, bias_f, scale):
    return pl.pallas_call(
        _pass_b_body,
        grid=(N // BLK,),
        in_specs=[
            pl.BlockSpec((MH, BLK, F2), lambda i: (0, i, 0)),
            pl.BlockSpec((MH, BLK, F2), lambda i: (0, i, 0)),
            pl.BlockSpec((M, BLK, NBR), lambda i: (0, i, 0)),
            pl.BlockSpec((BLK, F), lambda i: (i, 0)),
            pl.BlockSpec((F, F2), lambda i: (0, 0)),
            pl.BlockSpec((NBR, F2), lambda i: (0, 0)),
            pl.BlockSpec((1, F2), lambda i: (0, 0)),
            pl.BlockSpec((1, F2), lambda i: (0, 0)),
        ],
        out_specs=[
            pl.BlockSpec((BLK, F), lambda i: (i, 0)),
            pl.BlockSpec((8, F2), lambda i: (0, 0)),
        ],
        out_shape=[
            jax.ShapeDtypeStruct((N, F), jnp.float32),
            jax.ShapeDtypeStruct((8, F2), jnp.float32),
        ],
    )(g1, g2, e, x, Ws_f, We_f, bias_f.reshape(1, F2), scale.reshape(1, F2))


def _update_body(x_ref, sm_ref, a_ref, c_ref, wn_ref, o_ref, y_ref):
    xn = jax.nn.softplus(x_ref[...] + sm_ref[...] * a_ref[...] + c_ref[...])
    o_ref[...] = xn
    y_ref[...] = _dot(xn, wn_ref[...])


def _update(x, summed, a, c, Wn_next):
    return pl.pallas_call(
        _update_body,
        grid=(N // 1000,),
        in_specs=[
            pl.BlockSpec((1000, F), lambda i: (i, 0)),
            pl.BlockSpec((1000, F), lambda i: (i, 0)),
            pl.BlockSpec((1, F), lambda i: (0, 0)),
            pl.BlockSpec((1, F), lambda i: (0, 0)),
            pl.BlockSpec((F, F2), lambda i: (0, 0)),
        ],
        out_specs=[
            pl.BlockSpec((1000, F), lambda i: (i, 0)),
            pl.BlockSpec((1000, F2), lambda i: (i, 0)),
        ],
        out_shape=[
            jax.ShapeDtypeStruct((N, F), jnp.float32),
            jax.ShapeDtypeStruct((N, F2), jnp.float32),
        ],
    )(x, summed, a.reshape(1, F), c.reshape(1, F), Wn_next)


def _update_head_body(x_ref, sm_ref, a_ref, c_ref, wh_ref, bh_ref, o_ref):
    xn = jax.nn.softplus(x_ref[...] + sm_ref[...] * a_ref[...] + c_ref[...])
    o_ref[...] = _dot(xn, wh_ref[...]) + bh_ref[...]


def _update_head(x, summed, a, c, W_head, b_head):
    H = W_head.shape[1]
    return pl.pallas_call(
        _update_head_body,
        grid=(N // 1000,),
        in_specs=[
            pl.BlockSpec((1000, F), lambda i: (i, 0)),
            pl.BlockSpec((1000, F), lambda i: (i, 0)),
            pl.BlockSpec((1, F), lambda i: (0, 0)),
            pl.BlockSpec((1, F), lambda i: (0, 0)),
            pl.BlockSpec((F, H), lambda i: (0, 0)),
            pl.BlockSpec((1, H), lambda i: (0, 0)),
        ],
        out_specs=pl.BlockSpec((1000, H), lambda i: (i, 0)),
        out_shape=jax.ShapeDtypeStruct((N, H), jnp.float32),
    )(x, summed, a.reshape(1, F), c.reshape(1, F), W_head,
      b_head.reshape(1, H))


def _conv_layer(x, y, idx1, idx2, e, Wf, bf, g1, b1, g2n, b2n):
    Ws, We = Wf[:F], Wf[F2:]
    We_bf = We.astype(jnp.bfloat16)
    ga = _sc_gather(y, idx1).reshape(MH, N, F2)
    sta = _pass_a(ga, e, x, Ws, We_bf, bf, 0)
    gb = _sc_gather(y, idx2).reshape(MH, N, F2)
    stb = _pass_a(gb, e, x, Ws, We_bf, bf, 1)
    st = sta + stb

    mean = st[0, :F2] / (N * M)
    var = st[0, F2:] / (N * M) - mean * mean
    scale = g1 * lax.rsqrt(var + 1e-5)
    shift = b1 - mean * scale

    Ws_f = Ws * scale
    We_f = We * scale
    bias_f = bf * scale + shift

    summed, st2 = _pass_b(ga, gb, e, x, Ws_f, We_f.astype(jnp.bfloat16),
                          bias_f, scale)
    m2 = st2[0, :F] / N
    v2 = st2[0, F:] / N - m2 * m2
    a = g2n * lax.rsqrt(v2 + 1e-5)
    c = b2n - m2 * a
    return summed, a, c


def kernel(atom_fea, nbr_fea, nbr_fea_idx, crystal_atom_idx, W_emb, b_emb,
           conv0_Wf, conv0_bf, conv0_g1, conv0_b1, conv0_g2, conv0_b2,
           conv1_Wf, conv1_bf, conv1_g1, conv1_b1, conv1_g2, conv1_b2,
           conv2_Wf, conv2_bf, conv2_g1, conv2_b1, conv2_g2, conv2_b2,
           W_head, b_head):
    del crystal_atom_idx
    idx_t = nbr_fea_idx.T
    idx1 = idx_t[:MH].reshape(N * MH)
    idx2 = idx_t[MH:].reshape(N * MH)
    e = nbr_fea.transpose(1, 0, 2).astype(jnp.bfloat16)

    x, y = _embed(atom_fea, W_emb, b_emb, conv0_Wf[F:F2])
    params = [
        (conv0_Wf, conv0_bf, conv0_g1, conv0_b1, conv0_g2, conv0_b2),
        (conv1_Wf, conv1_bf, conv1_g1, conv1_b1, conv1_g2, conv1_b2),
        (conv2_Wf, conv2_bf, conv2_g1, conv2_b1, conv2_g2, conv2_b2),
    ]
    for l, p in enumerate(params):
        summed, a, c = _conv_layer(x, y, idx1, idx2, e, *p)
        if l < 2:
            x, y = _update(x, summed, a, c, params[l + 1][0][F:F2])
        else:
            return _update_head(x, summed, a, c, W_head, b_head)


# speedup vs baseline: 1.1082x; 1.0557x over previous
import functools

import jax
import jax.numpy as jnp
from jax import lax
from jax.experimental import pallas as pl
from jax.experimental.pallas import tpu as pltpu
from jax.experimental.pallas import tpu_sc as plsc

N, M = 10000, 32
ORIG, NBR, F = 128, 16, 64
F2 = 2 * F

BLK = 400
MH = M // 2
ROWSH = BLK * MH
GCH = 1000

_PREC = lax.Precision.DEFAULT


def _dot(a, b):
    return lax.dot_general(a, b, (((1,), (0,)), ((), ())),
                           precision=_PREC, preferred_element_type=jnp.float32)


def _sc_gather(table, idx_flat):
    n_idx = idx_flat.shape[0]
    mesh = plsc.VectorSubcoreMesh(core_axis_name="c", subcore_axis_name="s")
    n_workers = 32
    per_w = n_idx // n_workers

    @functools.partial(
        pl.kernel,
        mesh=mesh,
        out_type=jax.ShapeDtypeStruct((n_idx, F2), jnp.float32),
        scratch_types=[
            pltpu.VMEM((GCH,), jnp.int32),
            pltpu.VMEM((GCH, F2), jnp.float32),
            pltpu.SemaphoreType.DMA,
        ],
    )
    def gk(table_hbm, idx_hbm, out_hbm, idx_v, rows_v, sem):
        wid = lax.axis_index("s") * 2 + lax.axis_index("c")
        base = wid * per_w

        @pl.loop(0, per_w, step=GCH)
        def _(off):
            pltpu.sync_copy(idx_hbm.at[pl.ds(base + off, GCH)], idx_v)
            pltpu.async_copy(table_hbm.at[idx_v], rows_v, sem).wait()
            pltpu.sync_copy(rows_v, out_hbm.at[pl.ds(base + off, GCH)])

    return gk(table, idx_flat)


def _embed_body(a_ref, w_ref, b_ref, wn_ref, x_ref, y_ref):
    x = _dot(a_ref[...], w_ref[...]) + b_ref[...]
    x_ref[...] = x
    y_ref[...] = _dot(x, wn_ref[...])


def _embed(atom_fea, W_emb, b_emb, Wn0):
    return pl.pallas_call(
        _embed_body,
        grid=(N // 1000,),
        in_specs=[
            pl.BlockSpec((1000, ORIG), lambda i: (i, 0)),
            pl.BlockSpec((ORIG, F), lambda i: (0, 0)),
            pl.BlockSpec((1, F), lambda i: (0, 0)),
            pl.BlockSpec((F, F2), lambda i: (0, 0)),
        ],
        out_specs=[
            pl.BlockSpec((1000, F), lambda i: (i, 0)),
            pl.BlockSpec((1000, F2), lambda i: (i, 0)),
        ],
        out_shape=[
            jax.ShapeDtypeStruct((N, F), jnp.float32),
            jax.ShapeDtypeStruct((N, F2), jnp.float32),
        ],
    )(atom_fea, W_emb, b_emb.reshape(1, F), Wn0)


def _pass_a_body(g_ref, e_ref, x_ref, ws_ref, we_ref, bf_ref, st_ref):
    i = pl.program_id(0)
    s = _dot(x_ref[...], ws_ref[...]) + bf_ref[...]
    eterm = _dot(e_ref[...].reshape(ROWSH, NBR), we_ref[...])
    gated = (g_ref[...].reshape(ROWSH, F2) + eterm).reshape(MH, BLK, F2)
    gated = gated + s[None, :, :]
    flat = gated.reshape(ROWSH, F2)
    both = jnp.concatenate([flat, flat * flat], axis=1)
    upd = _dot(jnp.ones((8, ROWSH), jnp.float32), both)

    @pl.when(i == 0)
    def _():
        st_ref[...] = upd

    @pl.when(i > 0)
    def _():
        st_ref[...] = st_ref[...] + upd


def _pass_a(g_half, e, x, Ws, We, bf, mo):
    return pl.pallas_call(
        _pass_a_body,
        grid=(N // BLK,),
        in_specs=[
            pl.BlockSpec((MH, BLK, F2), lambda i: (0, i, 0)),
            pl.BlockSpec((MH, BLK, NBR), lambda i: (mo, i, 0)),
            pl.BlockSpec((BLK, F), lambda i: (i, 0)),
            pl.BlockSpec((F, F2), lambda i: (0, 0)),
            pl.BlockSpec((NBR, F2), lambda i: (0, 0)),
            pl.BlockSpec((1, F2), lambda i: (0, 0)),
        ],
        out_specs=pl.BlockSpec((8, 2 * F2), lambda i: (0, 0)),
        out_shape=jax.ShapeDtypeStruct((8, 2 * F2), jnp.float32),
    )(g_half, e, x, Ws, We, bf.reshape(1, F2))


def _pass_b_body(g1_ref, g2_ref, e_ref, x_ref, ws_ref, we_ref, bias_ref,
                 scale_ref, sum_ref, st_ref):
    i = pl.program_id(0)
    s = _dot(x_ref[...], ws_ref[...]) + bias_ref[...]
    ef = e_ref[...]

    def half(g_half, e_half):
        eterm = _dot(e_half.reshape(ROWSH, NBR), we_ref[...])
        gated = (g_half.reshape(ROWSH, F2) * scale_ref[...]
                 + eterm).reshape(MH, BLK, F2)
        gated = (gated + s[None, :, :]).astype(jnp.bfloat16)
        u = gated[..., :F]
        v = gated[..., F:]
        sig = 0.5 * jnp.tanh(0.5 * u) + 0.5
        sp = jnp.maximum(v, 0.0) + jnp.log(1.0 + jnp.exp(-jnp.abs(v)))
        prod = (sig * sp).astype(jnp.float32)
        return jnp.sum(prod, axis=0)

    summed = half(g1_ref[...], ef[:MH]) + half(g2_ref[...], ef[MH:])
    sum_ref[...] = summed
    both = jnp.concatenate([summed, summed * summed], axis=1)
    upd = _dot(jnp.ones((8, BLK), jnp.float32), both)

    @pl.when(i == 0)
    def _():
        st_ref[...] = upd

    @pl.when(i > 0)
    def _():
        st_ref[...] = st_ref[...] + upd


def _pass_b(g1, g2, e, x, Ws_f, We_f, bias_f, scale):
    return pl.pallas_call(
        _pass_b_body,
        grid=(N // BLK,),
        in_specs=[
            pl.BlockSpec((MH, BLK, F2), lambda i: (0, i, 0)),
            pl.BlockSpec((MH, BLK, F2), lambda i: (0, i, 0)),
            pl.BlockSpec((M, BLK, NBR), lambda i: (0, i, 0)),
            pl.BlockSpec((BLK, F), lambda i: (i, 0)),
            pl.BlockSpec((F, F2), lambda i: (0, 0)),
            pl.BlockSpec((NBR, F2), lambda i: (0, 0)),
            pl.BlockSpec((1, F2), lambda i: (0, 0)),
            pl.BlockSpec((1, F2), lambda i: (0, 0)),
        ],
        out_specs=[
            pl.BlockSpec((BLK, F), lambda i: (i, 0)),
            pl.BlockSpec((8, F2), lambda i: (0, 0)),
        ],
        out_shape=[
            jax.ShapeDtypeStruct((N, F), jnp.float32),
            jax.ShapeDtypeStruct((8, F2), jnp.float32),
        ],
    )(g1, g2, e, x, Ws_f, We_f, bias_f.reshape(1, F2), scale.reshape(1, F2))


def _update_body(x_ref, sm_ref, a_ref, c_ref, wn_ref, o_ref, y_ref):
    xn = jax.nn.softplus(x_ref[...] + sm_ref[...] * a_ref[...] + c_ref[...])
    o_ref[...] = xn
    y_ref[...] = _dot(xn, wn_ref[...])


def _update(x, summed, a, c, Wn_next):
    return pl.pallas_call(
        _update_body,
        grid=(N // 1000,),
        in_specs=[
            pl.BlockSpec((1000, F), lambda i: (i, 0)),
            pl.BlockSpec((1000, F), lambda i: (i, 0)),
            pl.BlockSpec((1, F), lambda i: (0, 0)),
            pl.BlockSpec((1, F), lambda i: (0, 0)),
            pl.BlockSpec((F, F2), lambda i: (0, 0)),
        ],
        out_specs=[
            pl.BlockSpec((1000, F), lambda i: (i, 0)),
            pl.BlockSpec((1000, F2), lambda i: (i, 0)),
        ],
        out_shape=[
            jax.ShapeDtypeStruct((N, F), jnp.float32),
            jax.ShapeDtypeStruct((N, F2), jnp.float32),
        ],
    )(x, summed, a.reshape(1, F), c.reshape(1, F), Wn_next)


def _update_head_body(x_ref, sm_ref, a_ref, c_ref, wh_ref, bh_ref, o_ref):
    xn = jax.nn.softplus(x_ref[...] + sm_ref[...] * a_ref[...] + c_ref[...])
    o_ref[...] = _dot(xn, wh_ref[...]) + bh_ref[...]


def _update_head(x, summed, a, c, W_head, b_head):
    H = W_head.shape[1]
    return pl.pallas_call(
        _update_head_body,
        grid=(N // 1000,),
        in_specs=[
            pl.BlockSpec((1000, F), lambda i: (i, 0)),
            pl.BlockSpec((1000, F), lambda i: (i, 0)),
            pl.BlockSpec((1, F), lambda i: (0, 0)),
            pl.BlockSpec((1, F), lambda i: (0, 0)),
            pl.BlockSpec((F, H), lambda i: (0, 0)),
            pl.BlockSpec((1, H), lambda i: (0, 0)),
        ],
        out_specs=pl.BlockSpec((1000, H), lambda i: (i, 0)),
        out_shape=jax.ShapeDtypeStruct((N, H), jnp.float32),
    )(x, summed, a.reshape(1, F), c.reshape(1, F), W_head,
      b_head.reshape(1, H))


def _conv_layer(x, y, idx1, idx2, e, Wf, bf, g1, b1, g2n, b2n):
    Ws, We = Wf[:F], Wf[F2:]
    We_bf = We.astype(jnp.bfloat16)
    ga = _sc_gather(y, idx1).reshape(MH, N, F2)
    sta = _pass_a(ga, e, x, Ws, We_bf, bf, 0)
    gb = _sc_gather(y, idx2).reshape(MH, N, F2)
    stb = _pass_a(gb, e, x, Ws, We_bf, bf, 1)
    st = sta + stb

    mean = st[0, :F2] / (N * M)
    var = st[0, F2:] / (N * M) - mean * mean
    scale = g1 * lax.rsqrt(var + 1e-5)
    shift = b1 - mean * scale

    Ws_f = Ws * scale
    We_f = We * scale
    bias_f = bf * scale + shift

    summed, st2 = _pass_b(ga, gb, e, x, Ws_f, We_f.astype(jnp.bfloat16),
                          bias_f, scale)
    m2 = st2[0, :F] / N
    v2 = st2[0, F:] / N - m2 * m2
    a = g2n * lax.rsqrt(v2 + 1e-5)
    c = b2n - m2 * a
    return summed, a, c


def kernel(atom_fea, nbr_fea, nbr_fea_idx, crystal_atom_idx, W_emb, b_emb,
           conv0_Wf, conv0_bf, conv0_g1, conv0_b1, conv0_g2, conv0_b2,
           conv1_Wf, conv1_bf, conv1_g1, conv1_b1, conv1_g2, conv1_b2,
           conv2_Wf, conv2_bf, conv2_g1, conv2_b1, conv2_g2, conv2_b2,
           W_head, b_head):
    del crystal_atom_idx
    idx_t = nbr_fea_idx.T
    idx1 = idx_t[:MH].reshape(N * MH)
    idx2 = idx_t[MH:].reshape(N * MH)
    e = nbr_fea.transpose(1, 0, 2).astype(jnp.bfloat16)

    x, y = _embed(atom_fea, W_emb, b_emb, conv0_Wf[F:F2])
    params = [
        (conv0_Wf, conv0_bf, conv0_g1, conv0_b1, conv0_g2, conv0_b2),
        (conv1_Wf, conv1_bf, conv1_g1, conv1_b1, conv1_g2, conv1_b2),
        (conv2_Wf, conv2_bf, conv2_g1, conv2_b1, conv2_g2, conv2_b2),
    ]
    for l, p in enumerate(params):
        summed, a, c = _conv_layer(x, y, idx1, idx2, e, *p)
        if l < 2:
            x, y = _update(x, summed, a, c, params[l + 1][0][F:F2])
        else:
            return _update_head(x, summed, a, c, W_head, b_head)


# speedup vs baseline: 1.1355x; 1.0247x over previous
import functools

import jax
import jax.numpy as jnp
from jax import lax
from jax.experimental import pallas as pl
from jax.experimental.pallas import tpu as pltpu
from jax.experimental.pallas import tpu_sc as plsc

N, M = 10000, 32
ORIG, NBR, F = 128, 16, 64
F2 = 2 * F

BLK = 400
ROWSF = BLK * M
GCH = 1000

_PREC = lax.Precision.DEFAULT


def _dot(a, b):
    return lax.dot_general(a, b, (((1,), (0,)), ((), ())),
                           precision=_PREC, preferred_element_type=jnp.float32)


def _sc_gather(table, idx_flat):
    n_idx = idx_flat.shape[0]
    mesh = plsc.VectorSubcoreMesh(core_axis_name="c", subcore_axis_name="s")
    n_workers = 32
    per_w = n_idx // n_workers

    @functools.partial(
        pl.kernel,
        mesh=mesh,
        out_type=jax.ShapeDtypeStruct((n_idx, F2), jnp.float32),
        scratch_types=[
            pltpu.VMEM((GCH,), jnp.int32),
            pltpu.VMEM((GCH, F2), jnp.float32),
            pltpu.SemaphoreType.DMA,
        ],
    )
    def gk(table_hbm, idx_hbm, out_hbm, idx_v, rows_v, sem):
        wid = lax.axis_index("s") * 2 + lax.axis_index("c")
        base = wid * per_w

        @pl.loop(0, per_w, step=GCH)
        def _(off):
            pltpu.sync_copy(idx_hbm.at[pl.ds(base + off, GCH)], idx_v)
            pltpu.async_copy(table_hbm.at[idx_v], rows_v, sem).wait()
            pltpu.sync_copy(rows_v, out_hbm.at[pl.ds(base + off, GCH)])

    return gk(table, idx_flat)


def _embed_body(a_ref, w_ref, b_ref, wn_ref, x_ref, y_ref):
    x = _dot(a_ref[...], w_ref[...]) + b_ref[...]
    x_ref[...] = x
    y_ref[...] = _dot(x, wn_ref[...])


def _embed(atom_fea, W_emb, b_emb, Wn0):
    return pl.pallas_call(
        _embed_body,
        grid=(N // 1000,),
        in_specs=[
            pl.BlockSpec((1000, ORIG), lambda i: (i, 0)),
            pl.BlockSpec((ORIG, F), lambda i: (0, 0)),
            pl.BlockSpec((1, F), lambda i: (0, 0)),
            pl.BlockSpec((F, F2), lambda i: (0, 0)),
        ],
        out_specs=[
            pl.BlockSpec((1000, F), lambda i: (i, 0)),
            pl.BlockSpec((1000, F2), lambda i: (i, 0)),
        ],
        out_shape=[
            jax.ShapeDtypeStruct((N, F), jnp.float32),
            jax.ShapeDtypeStruct((N, F2), jnp.float32),
        ],
    )(atom_fea, W_emb, b_emb.reshape(1, F), Wn0)


def _pass_a_body(g_ref, e_ref, x_ref, ws_ref, we_ref, bf_ref, st_ref):
    i = pl.program_id(0)
    s = _dot(x_ref[...], ws_ref[...]) + bf_ref[...]
    eterm = _dot(e_ref[...].reshape(ROWSF, NBR), we_ref[...])
    gated = (g_ref[...].reshape(ROWSF, F2) + eterm).reshape(M, BLK, F2)
    gated = gated + s[None, :, :]
    flat = gated.reshape(ROWSF, F2).astype(jnp.bfloat16)
    both = jnp.concatenate([flat, flat * flat], axis=1)
    upd = _dot(jnp.ones((8, ROWSF), jnp.bfloat16), both)

    @pl.when(i == 0)
    def _():
        st_ref[...] = upd

    @pl.when(i > 0)
    def _():
        st_ref[...] = st_ref[...] + upd


def _pass_a(g, e, x, Ws, We, bf):
    return pl.pallas_call(
        _pass_a_body,
        grid=(N // BLK,),
        in_specs=[
            pl.BlockSpec((M, BLK, F2), lambda i: (0, i, 0)),
            pl.BlockSpec((M, BLK, NBR), lambda i: (0, i, 0)),
            pl.BlockSpec((BLK, F), lambda i: (i, 0)),
            pl.BlockSpec((F, F2), lambda i: (0, 0)),
            pl.BlockSpec((NBR, F2), lambda i: (0, 0)),
            pl.BlockSpec((1, F2), lambda i: (0, 0)),
        ],
        out_specs=pl.BlockSpec((8, 2 * F2), lambda i: (0, 0)),
        out_shape=jax.ShapeDtypeStruct((8, 2 * F2), jnp.float32),
    )(g, e, x, Ws, We, bf.reshape(1, F2))


def _pass_b_body(g_ref, e_ref, x_ref, ws_ref, we_ref, bias_ref,
                 scale_ref, sum_ref, st_ref):
    i = pl.program_id(0)
    s = _dot(x_ref[...], ws_ref[...]) + bias_ref[...]
    eterm = _dot(e_ref[...].reshape(ROWSF, NBR), we_ref[...])
    gated = (g_ref[...].reshape(ROWSF, F2) * scale_ref[...]
             + eterm).reshape(M, BLK, F2)
    gated = (gated + s[None, :, :]).astype(jnp.bfloat16)
    u = gated[..., :F]
    v = gated[..., F:]
    sig = 0.5 * jnp.tanh(0.5 * u) + 0.5
    sp = jnp.maximum(v, 0.0) + jnp.log(1.0 + jnp.exp(-jnp.abs(v)))
    prod = (sig * sp).astype(jnp.float32)
    summed = jnp.sum(prod, axis=0)
    sum_ref[...] = summed
    both = jnp.concatenate([summed, summed * summed], axis=1)
    upd = _dot(jnp.ones((8, BLK), jnp.float32), both)

    @pl.when(i == 0)
    def _():
        st_ref[...] = upd

    @pl.when(i > 0)
    def _():
        st_ref[...] = st_ref[...] + upd


def _pass_b(g, e, x, Ws_f, We_f, bias_f, scale):
    return pl.pallas_call(
        _pass_b_body,
        grid=(N // BLK,),
        in_specs=[
            pl.BlockSpec((M, BLK, F2), lambda i: (0, i, 0)),
            pl.BlockSpec((M, BLK, NBR), lambda i: (0, i, 0)),
            pl.BlockSpec((BLK, F), lambda i: (i, 0)),
            pl.BlockSpec((F, F2), lambda i: (0, 0)),
            pl.BlockSpec((NBR, F2), lambda i: (0, 0)),
            pl.BlockSpec((1, F2), lambda i: (0, 0)),
            pl.BlockSpec((1, F2), lambda i: (0, 0)),
        ],
        out_specs=[
            pl.BlockSpec((BLK, F), lambda i: (i, 0)),
            pl.BlockSpec((8, F2), lambda i: (0, 0)),
        ],
        out_shape=[
            jax.ShapeDtypeStruct((N, F), jnp.float32),
            jax.ShapeDtypeStruct((8, F2), jnp.float32),
        ],
    )(g, e, x, Ws_f, We_f, bias_f.reshape(1, F2), scale.reshape(1, F2))


def _update_body(x_ref, sm_ref, a_ref, c_ref, wn_ref, o_ref, y_ref):
    xn = jax.nn.softplus(x_ref[...] + sm_ref[...] * a_ref[...] + c_ref[...])
    o_ref[...] = xn
    y_ref[...] = _dot(xn, wn_ref[...])


def _update(x, summed, a, c, Wn_next):
    return pl.pallas_call(
        _update_body,
        grid=(N // 1000,),
        in_specs=[
            pl.BlockSpec((1000, F), lambda i: (i, 0)),
            pl.BlockSpec((1000, F), lambda i: (i, 0)),
            pl.BlockSpec((1, F), lambda i: (0, 0)),
            pl.BlockSpec((1, F), lambda i: (0, 0)),
            pl.BlockSpec((F, F2), lambda i: (0, 0)),
        ],
        out_specs=[
            pl.BlockSpec((1000, F), lambda i: (i, 0)),
            pl.BlockSpec((1000, F2), lambda i: (i, 0)),
        ],
        out_shape=[
            jax.ShapeDtypeStruct((N, F), jnp.float32),
            jax.ShapeDtypeStruct((N, F2), jnp.float32),
        ],
    )(x, summed, a.reshape(1, F), c.reshape(1, F), Wn_next)


def _update_head_body(x_ref, sm_ref, a_ref, c_ref, wh_ref, bh_ref, o_ref):
    xn = jax.nn.softplus(x_ref[...] + sm_ref[...] * a_ref[...] + c_ref[...])
    o_ref[...] = _dot(xn, wh_ref[...]) + bh_ref[...]


def _update_head(x, summed, a, c, W_head, b_head):
    H = W_head.shape[1]
    return pl.pallas_call(
        _update_head_body,
        grid=(N // 1000,),
        in_specs=[
            pl.BlockSpec((1000, F), lambda i: (i, 0)),
            pl.BlockSpec((1000, F), lambda i: (i, 0)),
            pl.BlockSpec((1, F), lambda i: (0, 0)),
            pl.BlockSpec((1, F), lambda i: (0, 0)),
            pl.BlockSpec((F, H), lambda i: (0, 0)),
            pl.BlockSpec((1, H), lambda i: (0, 0)),
        ],
        out_specs=pl.BlockSpec((1000, H), lambda i: (i, 0)),
        out_shape=jax.ShapeDtypeStruct((N, H), jnp.float32),
    )(x, summed, a.reshape(1, F), c.reshape(1, F), W_head,
      b_head.reshape(1, H))


def _conv_layer(x, y, idx_flat, e, Wf, bf, g1, b1, g2n, b2n):
    Ws, We = Wf[:F], Wf[F2:]
    g = _sc_gather(y, idx_flat).reshape(M, N, F2)

    st = _pass_a(g, e, x, Ws, We.astype(jnp.bfloat16), bf)
    mean = st[0, :F2] / (N * M)
    var = st[0, F2:] / (N * M) - mean * mean
    scale = g1 * lax.rsqrt(var + 1e-5)
    shift = b1 - mean * scale

    Ws_f = Ws * scale
    We_f = We * scale
    bias_f = bf * scale + shift

    summed, st2 = _pass_b(g, e, x, Ws_f, We_f.astype(jnp.bfloat16),
                          bias_f, scale)
    m2 = st2[0, :F] / N
    v2 = st2[0, F:] / N - m2 * m2
    a = g2n * lax.rsqrt(v2 + 1e-5)
    c = b2n - m2 * a
    return summed, a, c


def kernel(atom_fea, nbr_fea, nbr_fea_idx, crystal_atom_idx, W_emb, b_emb,
           conv0_Wf, conv0_bf, conv0_g1, conv0_b1, conv0_g2, conv0_b2,
           conv1_Wf, conv1_bf, conv1_g1, conv1_b1, conv1_g2, conv1_b2,
           conv2_Wf, conv2_bf, conv2_g1, conv2_b1, conv2_g2, conv2_b2,
           W_head, b_head):
    del crystal_atom_idx
    idx_flat = nbr_fea_idx.T.reshape(N * M)
    e = nbr_fea.transpose(1, 0, 2).astype(jnp.bfloat16)

    x, y = _embed(atom_fea, W_emb, b_emb, conv0_Wf[F:F2])
    params = [
        (conv0_Wf, conv0_bf, conv0_g1, conv0_b1, conv0_g2, conv0_b2),
        (conv1_Wf, conv1_bf, conv1_g1, conv1_b1, conv1_g2, conv1_b2),
        (conv2_Wf, conv2_bf, conv2_g1, conv2_b1, conv2_g2, conv2_b2),
    ]
    for l, p in enumerate(params):
        summed, a, c = _conv_layer(x, y, idx_flat, e, *p)
        if l < 2:
            x, y = _update(x, summed, a, c, params[l + 1][0][F:F2])
        else:
            return _update_head(x, summed, a, c, W_head, b_head)
